# baseline (speedup 1.0000x reference)
import functools

import jax
import jax.numpy as jnp
from jax import lax
from jax.experimental import pallas as pl
from jax.experimental.pallas import tpu as pltpu
from jax.experimental.pallas import tpu_sc as plsc

_RC2 = 36.0
_EPS = 1e-12
_L = 16


def _tc_body(d_pl_ref, pos_ref, vrow_ref, vcol_ref,
             vec_ref, mask_ref, *, SCH):
    p = pos_ref[0]
    vrowb = vrow_ref[0] > 0.0
    vcolb = vcol_ref[0] > 0.0
    for k in range(SCH):
        dp = d_pl_ref[0, k]
        vx = dp[0:1, :] - p[:, 0:1]
        vy = dp[1:2, :] - p[:, 1:2]
        vz = dp[2:3, :] - p[:, 2:3]
        sod = vx * vx + vy * vy + vz * vz
        m = (sod < _RC2) & (sod > _EPS) & vrowb & vcolb
        mask_ref[0, k] = m
        vec_ref[0, k, 0] = jnp.where(m, vx, 0.0)
        vec_ref[0, k, 1] = jnp.where(m, vy, 0.0)
        vec_ref[0, k, 2] = jnp.where(m, vz, 0.0)


def _tc_sod_body(d_pl_ref, pos_ref, vrow_ref, vcol_ref, sod_in_ref, sod_ref):
    dp = d_pl_ref[0, 0]
    p = pos_ref[0]
    vx = dp[0:1, :] - p[:, 0:1]
    vy = dp[1:2, :] - p[:, 1:2]
    vz = dp[2:3, :] - p[:, 2:3]
    sod = vx * vx + vy * vy + vz * vz
    m = ((sod < _RC2) & (sod > _EPS)
         & (vrow_ref[0] > 0.0) & (vcol_ref[0] > 0.0))
    sod_ref[0, 0] = jnp.where(m, sod, 0.0)


def _make_sc_sod(B, S, N, f32, T0):
    info = plsc.get_sparse_core_info()
    NC, NS = info.num_cores, info.num_subcores
    NW = NC * NS
    ROWS = N // NW
    NCH = N // _L

    @functools.partial(
        pl.kernel,
        mesh=plsc.VectorSubcoreMesh(core_axis_name="c", subcore_axis_name="s"),
        out_type=jax.ShapeDtypeStruct((B, S, N, N), f32),
        scratch_types=[
            pltpu.VMEM((2, 3, N), f32),
            pltpu.VMEM((B, ROWS, 3, _L), f32),
            pltpu.VMEM((2, ROWS, N), f32),
            pltpu.SemaphoreType.DMA,
            pltpu.SemaphoreType.DMA,
        ],
    )
    def sc_sod(d_hbm, pos_hbm, sod_hbm,
               d_v, pos_v, out_v, sem_out, sem_in):
        wid = lax.axis_index("s") * NC + lax.axis_index("c")
        r0 = wid * ROWS
        pltpu.sync_copy(pos_hbm.at[:, pl.ds(r0, ROWS)], pos_v)
        pltpu.make_async_copy(d_hbm.at[0, 0], d_v.at[0], sem_in).start()

        def plane(t, carry):
            b = t // S
            s = t % S
            buf = t % 2

            pltpu.make_async_copy(
                d_hbm.at[0, 0], d_v.at[buf], sem_in).wait()

            @pl.when(t + 1 < T0)
            def _prefetch():
                bn = (t + 1) // S
                sn = (t + 1) % S
                pltpu.make_async_copy(
                    d_hbm.at[bn, sn], d_v.at[(t + 1) % 2], sem_in).start()

            @pl.when(t >= 2)
            def _wait_ring():
                pltpu.make_async_copy(
                    sod_hbm.at[0, 0, pl.ds(0, ROWS), :], out_v.at[buf],
                    sem_out).wait()

            JAM = 4

            def rowq(q, carry2):
                r = q * JAM
                px = [pos_v[b, r + k, 0] for k in range(JAM)]
                py = [pos_v[b, r + k, 1] for k in range(JAM)]
                pz = [pos_v[b, r + k, 2] for k in range(JAM)]
                for c in range(NCH):
                    dx = d_v[buf, 0, pl.ds(c * _L, _L)]
                    dy = d_v[buf, 1, pl.ds(c * _L, _L)]
                    dz = d_v[buf, 2, pl.ds(c * _L, _L)]
                    for k in range(JAM):
                        vx = dx - px[k]
                        vy = dy - py[k]
                        vz = dz - pz[k]
                        sod = vx * vx + vy * vy + vz * vz
                        m = (sod < _RC2) & (sod > _EPS)
                        out_v[buf, r + k, pl.ds(c * _L, _L)] = (
                            jnp.where(m, sod, 0.0))
                return carry2

            lax.fori_loop(0, ROWS // JAM, rowq, 0)
            pltpu.make_async_copy(
                out_v.at[buf], sod_hbm.at[b, s, pl.ds(r0, ROWS), :],
                sem_out).start()
            return carry

        lax.fori_loop(0, T0, plane, 0)
        for _ in range(2):
            pltpu.make_async_copy(
                sod_hbm.at[0, 0, pl.ds(0, ROWS), :], out_v.at[0],
                sem_out).wait()

    return sc_sod


@jax.jit
def kernel(pos, cel, sft_cel, ent):
    B, N, _ = pos.shape
    S = sft_cel.shape[0]
    f32 = pos.dtype

    sft_xyz = jnp.einsum('sk,bkl->bsl', sft_cel.astype(f32), cel)
    d = pos[:, None, :, :] + sft_xyz[:, :, None, :]
    d_pl = d.transpose(0, 1, 3, 2)
    validf = (ent > 0).astype(f32)
    vrow = validf.reshape(B, 1, N)
    vcol = validf.reshape(B, N, 1)

    pos_enc = jnp.where(validf[..., None] > 0.0, pos, 1e6)
    d_enc = (pos_enc[:, None, :, :] + sft_xyz[:, :, None, :]).transpose(0, 1, 3, 2)
    pos_rep = jnp.broadcast_to(pos_enc[..., None], (B, N, 3, _L))

    T0 = 18
    sod_sc = _make_sc_sod(B, S, N, f32, T0)(d_enc, pos_rep)

    SCH = 3
    grid = (B, S // SCH)
    vec_out, mask_out = pl.pallas_call(
        functools.partial(_tc_body, SCH=SCH),
        grid=grid,
        in_specs=[
            pl.BlockSpec((1, SCH, 3, N), lambda b, s: (b, s, 0, 0)),
            pl.BlockSpec((1, N, 3), lambda b, s: (b, 0, 0)),
            pl.BlockSpec((1, 1, N), lambda b, s: (b, 0, 0)),
            pl.BlockSpec((1, N, 1), lambda b, s: (b, 0, 0)),
        ],
        out_specs=[
            pl.BlockSpec((1, SCH, 3, N, N), lambda b, s: (b, s, 0, 0, 0)),
            pl.BlockSpec((1, SCH, N, N), lambda b, s: (b, s, 0, 0)),
        ],
        out_shape=[
            jax.ShapeDtypeStruct((B, S, 3, N, N), f32),
            jax.ShapeDtypeStruct((B, S, N, N), jnp.bool_),
        ],
    )(d_pl, pos, vrow, vcol)


    SREM = B * S - T0
    sod_out = pl.pallas_call(
        _tc_sod_body,
        grid=(SREM,),
        in_specs=[
            pl.BlockSpec((1, 1, 3, N),
                         lambda t: ((t + T0) // S, (t + T0) % S, 0, 0)),
            pl.BlockSpec((1, N, 3), lambda t: ((t + T0) // S, 0, 0)),
            pl.BlockSpec((1, 1, N), lambda t: ((t + T0) // S, 0, 0)),
            pl.BlockSpec((1, N, 1), lambda t: ((t + T0) // S, 0, 0)),
            pl.BlockSpec(memory_space=pl.ANY),
        ],
        out_specs=pl.BlockSpec((1, 1, N, N),
                               lambda t: ((t + T0) // S, (t + T0) % S, 0, 0)),
        out_shape=jax.ShapeDtypeStruct((B, S, N, N), f32),
        input_output_aliases={4: 0},
    )(d_pl, pos, vrow, vcol, sod_sc)


    return vec_out.transpose(0, 1, 3, 4, 2), sod_out, mask_out


# speedup vs baseline: 1.4221x; 1.4221x over previous
import functools

import jax
import jax.numpy as jnp
from jax.experimental import pallas as pl

_RC2 = 36.0
_EPS = 1e-12


def _body(d_pl_ref, pos_ref, vrow_ref, vcol_ref,
          vec_ref, sod_ref, mask_ref, *, SC):
    p = pos_ref[0]
    vrowb = vrow_ref[0] > 0.0
    vcolb = vcol_ref[0] > 0.0
    for k in range(SC):
        dp = d_pl_ref[0, k]
        vx = dp[0:1, :] - p[:, 0:1]
        vy = dp[1:2, :] - p[:, 1:2]
        vz = dp[2:3, :] - p[:, 2:3]
        sod = vx * vx + vy * vy + vz * vz
        m = (sod < _RC2) & (sod > _EPS) & vrowb & vcolb
        sod_ref[0, k] = jnp.where(m, sod, 0.0)
        mask_ref[0, k] = m
        vec_ref[0, k, 0] = jnp.where(m, vx, 0.0)
        vec_ref[0, k, 1] = jnp.where(m, vy, 0.0)
        vec_ref[0, k, 2] = jnp.where(m, vz, 0.0)


@jax.jit
def kernel(pos, cel, sft_cel, ent):
    B, N, _ = pos.shape
    S = sft_cel.shape[0]
    f32 = pos.dtype

    sft_xyz = jnp.einsum('sk,bkl->bsl', sft_cel.astype(f32), cel)
    d = pos[:, None, :, :] + sft_xyz[:, :, None, :]
    d_pl = d.transpose(0, 1, 3, 2)
    validf = (ent > 0).astype(f32)
    vrow = validf.reshape(B, 1, N)
    vcol = validf.reshape(B, N, 1)

    SC = 3
    grid = (B, S // SC)
    vec_out, sod_out, mask_out = pl.pallas_call(
        functools.partial(_body, SC=SC),
        grid=grid,
        in_specs=[
            pl.BlockSpec((1, SC, 3, N), lambda b, s: (b, s, 0, 0)),
            pl.BlockSpec((1, N, 3), lambda b, s: (b, 0, 0)),
            pl.BlockSpec((1, 1, N), lambda b, s: (b, 0, 0)),
            pl.BlockSpec((1, N, 1), lambda b, s: (b, 0, 0)),
        ],
        out_specs=[
            pl.BlockSpec((1, SC, 3, N, N), lambda b, s: (b, s, 0, 0, 0)),
            pl.BlockSpec((1, SC, N, N), lambda b, s: (b, s, 0, 0)),
            pl.BlockSpec((1, SC, N, N), lambda b, s: (b, s, 0, 0)),
        ],
        out_shape=[
            jax.ShapeDtypeStruct((B, S, 3, N, N), f32),
            jax.ShapeDtypeStruct((B, S, N, N), f32),
            jax.ShapeDtypeStruct((B, S, N, N), jnp.bool_),
        ],
    )(d_pl, pos, vrow, vcol)

    return vec_out.transpose(0, 1, 3, 4, 2), sod_out, mask_out
